# Initial kernel scaffold; baseline (speedup 1.0000x reference)
import functools

import jax
import jax.numpy as jnp
from jax import lax
from jax.experimental import pallas as pl
from jax.experimental.pallas import tpu as pltpu
from jax.experimental.pallas import tpu_sc as plsc

VOCAB = 1000000
EMBED_DIM = 32
BATCH = 16384
HIST = 50

NUM_CORES = 2
NUM_SUBCORES = 16
NUM_WORKERS = NUM_CORES * NUM_SUBCORES

B_FLAT = BATCH * HIST
B_PER_W = B_FLAT // NUM_WORKERS
CHUNK = 3200
N_CHUNKS = B_PER_W // CHUNK


@functools.partial(
    pl.kernel,
    mesh=plsc.VectorSubcoreMesh(core_axis_name="c", subcore_axis_name="s"),
    out_type=jax.ShapeDtypeStruct((B_FLAT, EMBED_DIM), jnp.float32),
    scratch_types=[
        pltpu.VMEM((CHUNK,), jnp.int32),
        pltpu.VMEM((CHUNK, EMBED_DIM), jnp.float32),
        pltpu.SemaphoreType.DMA,
    ],
)
def _gather_rows(ids_hbm, table_hbm, out_hbm, idx_v, rows_v, sem):
    wid = lax.axis_index("s") * NUM_CORES + lax.axis_index("c")
    base = wid * B_PER_W

    def chunk_body(i, carry):
        off = base + i * CHUNK
        pltpu.sync_copy(ids_hbm.at[pl.ds(off, CHUNK)], idx_v)
        pltpu.async_copy(table_hbm.at[idx_v], rows_v, sem).wait()
        pltpu.sync_copy(rows_v, out_hbm.at[pl.ds(off, CHUNK)])
        return carry

    lax.fori_loop(0, N_CHUNKS, chunk_body, 0)


def kernel(input_ids, table):
    flat_ids = input_ids.reshape(-1).astype(jnp.int32)
    out = _gather_rows(flat_ids, table)
    return out.reshape(input_ids.shape + (EMBED_DIM,))


# speedup vs baseline: 1.1097x; 1.1097x over previous
import functools

import jax
import jax.numpy as jnp
from jax import lax
from jax.experimental import pallas as pl
from jax.experimental.pallas import tpu as pltpu
from jax.experimental.pallas import tpu_sc as plsc

VOCAB = 1000000
EMBED_DIM = 32
BATCH = 16384
HIST = 50

NUM_CORES = 2
NUM_SUBCORES = 16
NUM_WORKERS = NUM_CORES * NUM_SUBCORES

B_FLAT = BATCH * HIST
B_PER_W = B_FLAT // NUM_WORKERS
CHUNK = 3200
N_CHUNKS = B_PER_W // CHUNK


@functools.partial(
    pl.kernel,
    mesh=plsc.VectorSubcoreMesh(core_axis_name="c", subcore_axis_name="s"),
    out_type=jax.ShapeDtypeStruct((B_FLAT, EMBED_DIM), jnp.float32),
    scratch_types=[
        pltpu.VMEM((CHUNK,), jnp.int32),
        pltpu.VMEM((CHUNK, EMBED_DIM), jnp.float32),
        pltpu.SemaphoreType.DMA,
    ],
    compiler_params=pltpu.CompilerParams(use_tc_tiling_on_sc=False),
)
def _gather_rows(ids_hbm, table_hbm, out_hbm, idx_v, rows_v, sem):
    wid = lax.axis_index("s") * NUM_CORES + lax.axis_index("c")
    base = wid * B_PER_W

    def chunk_body(i, carry):
        off = base + i * CHUNK
        pltpu.sync_copy(ids_hbm.at[pl.ds(off, CHUNK)], idx_v)
        pltpu.async_copy(table_hbm.at[idx_v], rows_v, sem).wait()
        pltpu.sync_copy(rows_v, out_hbm.at[pl.ds(off, CHUNK)])
        return carry

    lax.fori_loop(0, N_CHUNKS, chunk_body, 0)


def kernel(input_ids, table):
    flat_ids = input_ids.reshape(-1).astype(jnp.int32)
    out = _gather_rows(flat_ids, table)
    return out.reshape(input_ids.shape + (EMBED_DIM,))


# speedup vs baseline: 1.1128x; 1.0028x over previous
import functools

import jax
import jax.numpy as jnp
from jax import lax
from jax.experimental import pallas as pl
from jax.experimental.pallas import tpu as pltpu
from jax.experimental.pallas import tpu_sc as plsc

VOCAB = 1000000
EMBED_DIM = 32
BATCH = 16384
HIST = 50

NUM_CORES = 2
NUM_SUBCORES = 16
NUM_WORKERS = NUM_CORES * NUM_SUBCORES

B_FLAT = BATCH * HIST
B_PER_W = B_FLAT // NUM_WORKERS
CHUNK = 1600
N_CHUNKS = B_PER_W // CHUNK
NI = 4
NR = 2


@functools.partial(
    pl.kernel,
    mesh=plsc.VectorSubcoreMesh(core_axis_name="c", subcore_axis_name="s"),
    out_type=jax.ShapeDtypeStruct((B_FLAT, EMBED_DIM), jnp.float32),
    scratch_types=[
        pltpu.VMEM((NI, CHUNK), jnp.int32),
        pltpu.VMEM((NR, CHUNK, EMBED_DIM), jnp.float32),
        [pltpu.SemaphoreType.DMA] * NI,
        [pltpu.SemaphoreType.DMA] * NR,
        [pltpu.SemaphoreType.DMA] * NR,
    ],
    compiler_params=pltpu.CompilerParams(use_tc_tiling_on_sc=False),
)
def _gather_rows(ids_hbm, table_hbm, out_hbm, idx_v, rows_v, sa, sg, ss):
    wid = lax.axis_index("s") * NUM_CORES + lax.axis_index("c")
    base = wid * B_PER_W

    def idx_slice(i):
        return ids_hbm.at[pl.ds(base + i * CHUNK, CHUNK)]

    def out_slice(i):
        return out_hbm.at[pl.ds(base + i * CHUNK, CHUNK)]

    def a_copy(i):
        return pltpu.make_async_copy(idx_slice(i), idx_v.at[i % NI], sa[i % NI])

    def b_copy(i):
        return pltpu.make_async_copy(
            table_hbm.at[idx_v.at[i % NI]], rows_v.at[i % NR], sg[i % NR])

    def c_copy(i):
        return pltpu.make_async_copy(rows_v.at[i % NR], out_slice(i), ss[i % NR])

    for j in range(NI):
        a_copy(j).start()
    a_copy(0).wait()
    b_copy(0).start()
    for i in range(N_CHUNKS):
        if i + 1 < N_CHUNKS:
            if i >= 1:
                c_copy(i - 1).wait()
            a_copy(i + 1).wait()
            b_copy(i + 1).start()
        b_copy(i).wait()
        c_copy(i).start()
        if i + NI < N_CHUNKS:
            a_copy(i + NI).start()
    c_copy(N_CHUNKS - 2).wait()
    c_copy(N_CHUNKS - 1).wait()


def kernel(input_ids, table):
    flat_ids = input_ids.reshape(-1).astype(jnp.int32)
    out = _gather_rows(flat_ids, table)
    return out.reshape(input_ids.shape + (EMBED_DIM,))


# speedup vs baseline: 1.8056x; 1.6226x over previous
import functools

import jax
import jax.numpy as jnp
from jax import lax
from jax.experimental import pallas as pl
from jax.experimental.pallas import tpu as pltpu
from jax.experimental.pallas import tpu_sc as plsc

VOCAB = 1000000
EMBED_DIM = 32
BATCH = 16384
HIST = 50

NUM_CORES = 2
NUM_SUBCORES = 16
NUM_WORKERS = NUM_CORES * NUM_SUBCORES

ROWS_PER_W = BATCH // NUM_WORKERS
NB = 8
NBUF = 2 * NB
N_GROUPS = ROWS_PER_W // NB


@functools.partial(
    pl.kernel,
    mesh=plsc.VectorSubcoreMesh(core_axis_name="c", subcore_axis_name="s"),
    out_type=jax.ShapeDtypeStruct((BATCH, HIST, EMBED_DIM), jnp.float32),
    scratch_types=[
        pltpu.VMEM((ROWS_PER_W, HIST), jnp.int32),
        pltpu.VMEM((NBUF, HIST, EMBED_DIM), jnp.float32),
        pltpu.SemaphoreType.DMA,
        [pltpu.SemaphoreType.DMA] * 2,
        [pltpu.SemaphoreType.DMA] * 2,
    ],
    compiler_params=pltpu.CompilerParams(use_tc_tiling_on_sc=False),
)
def _embed_lookup(ids_hbm, table_hbm, out_hbm, ids_v, rows_v, sem_i, sg, ss):
    wid = lax.axis_index("s") * NUM_CORES + lax.axis_index("c")
    j0 = wid * ROWS_PER_W

    pltpu.async_copy(ids_hbm.at[pl.ds(j0, ROWS_PER_W)], ids_v, sem_i).wait()

    def gather_copy(r, buf, h):
        return pltpu.make_async_copy(
            table_hbm.at[ids_v.at[r]], rows_v.at[buf], sg[h])

    def store_copy(r, buf, h):
        return pltpu.make_async_copy(rows_v.at[buf], out_hbm.at[j0 + r], ss[h])

    def half_step(g, h_static, refill):
        base = h_static * NB
        r0 = g * NB
        for b in range(NB):
            gather_copy(r0 + b, base + b, h_static).wait()
        for b in range(NB):
            store_copy(r0 + b, base + b, h_static).start()
        for b in range(NB):
            store_copy(r0 + b, base + b, h_static).wait()
        if refill:
            rn = r0 + 2 * NB
            for b in range(NB):
                gather_copy(rn + b, base + b, h_static).start()

    for b in range(NB):
        gather_copy(b, b, 0).start()
    for b in range(NB):
        gather_copy(NB + b, NB + b, 1).start()

    def pair_body(k, carry):
        half_step(2 * k, 0, True)
        half_step(2 * k + 1, 1, True)
        return carry

    lax.fori_loop(0, N_GROUPS // 2 - 1, pair_body, 0)
    half_step(N_GROUPS - 2, 0, False)
    half_step(N_GROUPS - 1, 1, False)


def kernel(input_ids, table):
    return _embed_lookup(input_ids.astype(jnp.int32), table)
